# baseline (speedup 1.0000x reference)
import jax
import jax.numpy as jnp
from jax import lax
from jax.experimental import pallas as pl
from jax.experimental.pallas import tpu as pltpu
from jax.experimental.pallas import tpu_sc as plsc

_B, _N, _D = 1024, 1024, 32
_NC, _NS = 2, 16
_NW = _NC * _NS
_BLOCKS_PER_W = _B // _NW
_IDX_MINOR = 128
_HALF = _N // 2
_JP = _HALF // _IDX_MINOR
_NCHUNK = 2 * _BLOCKS_PER_W
_VEC = 1024


def _pos_body(g_ref, w_ref, b_ref, o_ref):
    p = (
        jnp.dot(g_ref[...], w_ref[...], preferred_element_type=jnp.float32)
        + b_ref[...]
    )
    o_ref[...] = p.T.reshape(4, 8, 8, 128).transpose(0, 2, 1, 3).reshape(-1)


def _sc_body(x_hbm, tab_hbm, pos_hbm, out_hbm,
             idx0, idx1, rows0, rows1, tr0, tr1, pos_v, g0, g1, o0, o1):
    c = lax.axis_index("c")
    s = lax.axis_index("s")
    wid = s * _NC + c
    base = wid * _BLOCKS_PER_W
    idx = (idx0, idx1)
    rows = (rows0, rows1)
    tr = (tr0, tr1)
    gsem = (g0, g1)
    osem = (o0, o1)
    pltpu.sync_copy(pos_hbm, pos_v)
    lane = lax.iota(jnp.int32, 16)

    def fire(t, buf):
        b = base + lax.shift_right_logical(t, 1)
        h = lax.bitwise_and(t, 1)
        pltpu.sync_copy(x_hbm.at[b, h], idx[buf])
        for j in range(_JP):
            pltpu.async_copy(
                tab_hbm.at[idx[buf].at[j]],
                rows[buf].at[pl.ds(j * _IDX_MINOR, _IDX_MINOR)],
                gsem[buf],
            )

    fire(0, 0)
    for t in range(_NCHUNK):
        cur = t % 2
        nxt = 1 - cur
        out_dummy = out_hbm.at[0].at[:, pl.ds(0, 4096)]
        if t + 1 < _NCHUNK:
            if t >= 1:
                pltpu.make_async_copy(tr[nxt], out_dummy, osem[nxt]).wait()
            fire(t + 1, nxt)
        pltpu.make_async_copy(tab_hbm.at[pl.ds(0, _HALF)],
                              rows[cur], gsem[cur]).wait()

        h = lax.bitwise_and(t, 1)
        pos_half = lax.shift_left(h, 12)

        @plsc.parallel_loop(0, _VEC, step=1, unroll=8)
        def transpose_add(k):
            ncv = lax.bitwise_and(k, 7)
            dr = lax.bitwise_and(lax.shift_right_logical(k, 3), 7)
            nb = lax.bitwise_and(lax.shift_right_logical(k, 6), 3)
            db = lax.shift_right_logical(k, 8)
            n0 = lax.shift_left(nb, 7) + lax.shift_left(ncv, 4)
            d = lax.shift_left(db, 3) + dr
            n_idx = n0 + lane
            d_idx = jnp.broadcast_to(d, (16,))
            g = plsc.load_gather(rows[cur], [n_idx, d_idx])
            col = lax.shift_left(lax.bitwise_and(k, 255), 4)
            pos_off = lax.shift_left(k, 4) + lax.shift_left(db, 12) + pos_half
            tr[cur][db, pl.ds(col, 16)] = g + pos_v[pl.ds(pos_off, 16)]

        b = base + lax.shift_right_logical(t, 1)
        pltpu.async_copy(
            tr[cur],
            out_hbm.at[b].at[:, pl.ds(lax.shift_left(h, 12), 4096)],
            osem[cur],
        )

    out_dummy = out_hbm.at[0].at[:, pl.ds(0, 4096)]
    pltpu.make_async_copy(tr[0], out_dummy, osem[0]).wait()
    pltpu.make_async_copy(tr[1], out_dummy, osem[1]).wait()


def kernel(x, tok_table, pos_W, pos_b, grid):
    g2 = grid.reshape(_N, 4)
    pos = pl.pallas_call(
        _pos_body,
        out_shape=jax.ShapeDtypeStruct((_N * _D,), jnp.float32),
    )(g2, pos_W.T, pos_b.reshape(1, _D))

    x4 = x.reshape(_B, 2, _JP, _IDX_MINOR)
    sc = pl.kernel(
        _sc_body,
        out_type=jax.ShapeDtypeStruct((_B, 4, 8192), jnp.float32),
        mesh=plsc.VectorSubcoreMesh(core_axis_name="c", subcore_axis_name="s"),
        compiler_params=pltpu.CompilerParams(
            use_tc_tiling_on_sc=False, needs_layout_passes=False),
        scratch_types=[
            pltpu.VMEM((_JP, _IDX_MINOR), jnp.int32),
            pltpu.VMEM((_JP, _IDX_MINOR), jnp.int32),
            pltpu.VMEM((_HALF, _D), jnp.float32),
            pltpu.VMEM((_HALF, _D), jnp.float32),
            pltpu.VMEM((4, 4096), jnp.float32),
            pltpu.VMEM((4, 4096), jnp.float32),
            pltpu.VMEM((_N * _D,), jnp.float32),
            pltpu.SemaphoreType.DMA,
            pltpu.SemaphoreType.DMA,
            pltpu.SemaphoreType.DMA,
            pltpu.SemaphoreType.DMA,
        ],
    )
    out5 = sc(x4, tok_table, pos)
    out = out5.reshape(_B, 4, 8, 8, 128).transpose(0, 2, 4, 1, 3)
    return out.reshape(_B, _N, _D)


# speedup vs baseline: 3.2216x; 3.2216x over previous
import jax
import jax.numpy as jnp
from jax import lax
from jax.experimental import pallas as pl
from jax.experimental.pallas import tpu as pltpu
from jax.experimental.pallas import tpu_sc as plsc

_B, _N, _D = 1024, 1024, 32
_NC, _NS = 2, 16
_NW = _NC * _NS
_BLOCKS_PER_W = _B // _NW
_IDX_MINOR = 128
_HALF = _N // 2
_JP = _HALF // _IDX_MINOR
_NCHUNK = 2 * _BLOCKS_PER_W
_PITCH = 133


def _pos_body(g_ref, w_ref, b_ref, o_ref):
    o_ref[...] = (
        jnp.dot(g_ref[...], w_ref[...], preferred_element_type=jnp.float32)
        + b_ref[...]
    )


def _sc_body(x_hbm, tab_hbm, pos_hbm, out_hbm,
             idx0, idx1, rows0, rows1, tr0, tr1, pos_v, g0, g1, o0, o1):
    c = lax.axis_index("c")
    s = lax.axis_index("s")
    wid = s * _NC + c
    base = wid * _BLOCKS_PER_W
    idx = (idx0, idx1)
    rows = (rows0, rows1)
    tr = (tr0, tr1)
    gsem = (g0, g1)
    osem = (o0, o1)
    pltpu.sync_copy(pos_hbm, pos_v)
    lane = lax.iota(jnp.int32, 16)
    db_lo = lax.shift_right_logical(lane, 3)
    db_hi = db_lo + 2
    dr_c = lax.bitwise_and(lane, 7)

    def fire(t, buf):
        b = base + lax.shift_right_logical(t, 1)
        h = lax.bitwise_and(t, 1)
        pltpu.sync_copy(x_hbm.at[b, h], idx[buf])
        for j in range(_JP):
            pltpu.async_copy(
                tab_hbm.at[idx[buf].at[j]],
                rows[buf].at[pl.ds(j * _IDX_MINOR, _IDX_MINOR)],
                gsem[buf],
            )

    out_dummy = out_hbm.at[0].at[:, pl.ds(0, 4)]
    fire(0, 0)

    def pair(i, carry):
        for j in range(2):
            cur = j
            nxt = 1 - j
            t = i * 2 + j

            @pl.when(t >= 1)
            def _wait_wb():
                pltpu.make_async_copy(
                    tr[nxt].at[:, :, :, pl.ds(0, _IDX_MINOR)],
                    out_dummy, osem[nxt]).wait()

            @pl.when(t + 1 < _NCHUNK)
            def _fire_next():
                fire(t + 1, nxt)

            pltpu.make_async_copy(tab_hbm.at[pl.ds(0, _HALF)],
                                  rows[cur], gsem[cur]).wait()

            h = lax.bitwise_and(t, 1)
            noff = lax.shift_left(h, 9)

            @plsc.parallel_loop(0, _HALF, step=1, unroll=8)
            def transpose_add(k):
                nb = lax.shift_right_logical(k, 7)
                nc = lax.bitwise_and(k, 127)
                nbv = jnp.broadcast_to(nb, (16,))
                ncv = jnp.broadcast_to(nc, (16,))
                ng = noff + k
                v0 = rows[cur][k, pl.ds(0, 16)] + pos_v[ng, pl.ds(0, 16)]
                v1 = rows[cur][k, pl.ds(16, 16)] + pos_v[ng, pl.ds(16, 16)]
                plsc.store_scatter(tr[cur], [db_lo, nbv, dr_c, ncv], v0)
                plsc.store_scatter(tr[cur], [db_hi, nbv, dr_c, ncv], v1)

            b = base + lax.shift_right_logical(t, 1)
            pltpu.async_copy(
                tr[cur].at[:, :, :, pl.ds(0, _IDX_MINOR)],
                out_hbm.at[b].at[:, pl.ds(lax.shift_left(h, 2), 4)],
                osem[cur],
            )
        return carry

    lax.fori_loop(0, _NCHUNK // 2, pair, 0)
    pltpu.make_async_copy(tr[1].at[:, :, :, pl.ds(0, _IDX_MINOR)],
                          out_dummy, osem[1]).wait()


def kernel(x, tok_table, pos_W, pos_b, grid):
    g2 = grid.reshape(_N, 4)
    pos = pl.pallas_call(
        _pos_body,
        out_shape=jax.ShapeDtypeStruct((_N, _D), jnp.float32),
    )(g2, pos_W.T, pos_b.reshape(1, _D))

    x4 = x.reshape(_B, 2, _JP, _IDX_MINOR)
    sc = pl.kernel(
        _sc_body,
        out_type=jax.ShapeDtypeStruct((_B, 4, 8, 8, _IDX_MINOR), jnp.float32),
        mesh=plsc.VectorSubcoreMesh(core_axis_name="c", subcore_axis_name="s"),
        compiler_params=pltpu.CompilerParams(
            use_tc_tiling_on_sc=False, needs_layout_passes=False),
        scratch_types=[
            pltpu.VMEM((_JP, _IDX_MINOR), jnp.int32),
            pltpu.VMEM((_JP, _IDX_MINOR), jnp.int32),
            pltpu.VMEM((_HALF, _D), jnp.float32),
            pltpu.VMEM((_HALF, _D), jnp.float32),
            pltpu.VMEM((4, 4, 8, _PITCH), jnp.float32),
            pltpu.VMEM((4, 4, 8, _PITCH), jnp.float32),
            pltpu.VMEM((_N, _D), jnp.float32),
            pltpu.SemaphoreType.DMA,
            pltpu.SemaphoreType.DMA,
            pltpu.SemaphoreType.DMA,
            pltpu.SemaphoreType.DMA,
        ],
    )
    out5 = sc(x4, tok_table, pos)
    out = out5.transpose(0, 2, 4, 1, 3)
    return out.reshape(_B, _N, _D)


# speedup vs baseline: 3.2423x; 1.0064x over previous
import jax
import jax.numpy as jnp
from jax import lax
from jax.experimental import pallas as pl
from jax.experimental.pallas import tpu as pltpu
from jax.experimental.pallas import tpu_sc as plsc

_B, _N, _D = 1024, 1024, 32
_NC, _NS = 2, 16
_NW = _NC * _NS
_BLOCKS_PER_W = _B // _NW
_IDX_MINOR = 128
_HALF = _N // 2
_JP = _HALF // _IDX_MINOR
_NCHUNK = 2 * _BLOCKS_PER_W
_PITCH = 133


def _pos_body(g_ref, w_ref, b_ref, o_ref):
    o_ref[...] = (
        jnp.dot(g_ref[...], w_ref[...], preferred_element_type=jnp.float32)
        + b_ref[...]
    )


def _sc_body(x_hbm, tab_hbm, pos_hbm, out_hbm,
             idx0, idx1, rows0, rows1, tr0, tr1, pos_v, g0, g1, o0, o1):
    c = lax.axis_index("c")
    s = lax.axis_index("s")
    wid = s * _NC + c
    base = wid * _BLOCKS_PER_W
    idx = (idx0, idx1)
    rows = (rows0, rows1)
    tr = (tr0, tr1)
    gsem = (g0, g1)
    osem = (o0, o1)
    pltpu.sync_copy(pos_hbm, pos_v)
    lane = lax.iota(jnp.int32, 16)
    db_lo = lax.shift_right_logical(lane, 3)
    db_hi = db_lo + 2
    dr_c = lax.bitwise_and(lane, 7)

    def fire(t, buf):
        b = base + lax.shift_right_logical(t, 1)
        h = lax.bitwise_and(t, 1)
        bb = lax.shift_right_logical(b, 3)
        br = lax.bitwise_and(b, 7)
        pltpu.sync_copy(
            x_hbm.at[bb, pl.ds(lax.shift_left(h, 2), _JP), br], idx[buf])
        for j in range(_JP):
            pltpu.async_copy(
                tab_hbm.at[idx[buf].at[j]],
                rows[buf].at[pl.ds(j * _IDX_MINOR, _IDX_MINOR)],
                gsem[buf],
            )

    out_dummy = out_hbm.at[0].at[:, pl.ds(0, 4)]
    fire(0, 0)

    def pair(i, carry):
        for j in range(2):
            cur = j
            nxt = 1 - j
            t = i * 2 + j

            @pl.when(t >= 1)
            def _wait_wb():
                pltpu.make_async_copy(
                    tr[nxt].at[:, :, :, pl.ds(0, _IDX_MINOR)],
                    out_dummy, osem[nxt]).wait()

            @pl.when(t + 1 < _NCHUNK)
            def _fire_next():
                fire(t + 1, nxt)

            pltpu.make_async_copy(tab_hbm.at[pl.ds(0, _HALF)],
                                  rows[cur], gsem[cur]).wait()

            h = lax.bitwise_and(t, 1)
            noff = lax.shift_left(h, 9)

            @plsc.parallel_loop(0, _HALF, step=1, unroll=8)
            def transpose_add(k):
                nb = lax.shift_right_logical(k, 7)
                nc = lax.bitwise_and(k, 127)
                nbv = jnp.broadcast_to(nb, (16,))
                ncv = jnp.broadcast_to(nc, (16,))
                ng = noff + k
                v0 = rows[cur][k, pl.ds(0, 16)] + pos_v[ng, pl.ds(0, 16)]
                v1 = rows[cur][k, pl.ds(16, 16)] + pos_v[ng, pl.ds(16, 16)]
                plsc.store_scatter(tr[cur], [db_lo, nbv, dr_c, ncv], v0)
                plsc.store_scatter(tr[cur], [db_hi, nbv, dr_c, ncv], v1)

            b = base + lax.shift_right_logical(t, 1)
            pltpu.async_copy(
                tr[cur].at[:, :, :, pl.ds(0, _IDX_MINOR)],
                out_hbm.at[b].at[:, pl.ds(lax.shift_left(h, 2), 4)],
                osem[cur],
            )
        return carry

    lax.fori_loop(0, _NCHUNK // 2, pair, 0)
    pltpu.make_async_copy(tr[1].at[:, :, :, pl.ds(0, _IDX_MINOR)],
                          out_dummy, osem[1]).wait()


def kernel(x, tok_table, pos_W, pos_b, grid):
    g2 = grid.reshape(_N, 4)
    pos = pl.pallas_call(
        _pos_body,
        out_shape=jax.ShapeDtypeStruct((_N, _D), jnp.float32),
    )(g2, pos_W.T, pos_b.reshape(1, _D))

    x4 = x.reshape(_B // 8, 8, 8, _IDX_MINOR).transpose(0, 2, 1, 3)
    tab_lin = lax.optimization_barrier(tok_table.T).T
    sc = pl.kernel(
        _sc_body,
        out_type=jax.ShapeDtypeStruct((_B, 4, 8, 8, _IDX_MINOR), jnp.float32),
        mesh=plsc.VectorSubcoreMesh(core_axis_name="c", subcore_axis_name="s"),
        compiler_params=pltpu.CompilerParams(
            use_tc_tiling_on_sc=False, needs_layout_passes=False),
        scratch_types=[
            pltpu.VMEM((_JP, _IDX_MINOR), jnp.int32),
            pltpu.VMEM((_JP, _IDX_MINOR), jnp.int32),
            pltpu.VMEM((_HALF, _D), jnp.float32),
            pltpu.VMEM((_HALF, _D), jnp.float32),
            pltpu.VMEM((4, 4, 8, _PITCH), jnp.float32),
            pltpu.VMEM((4, 4, 8, _PITCH), jnp.float32),
            pltpu.VMEM((_N, _D), jnp.float32),
            pltpu.SemaphoreType.DMA,
            pltpu.SemaphoreType.DMA,
            pltpu.SemaphoreType.DMA,
            pltpu.SemaphoreType.DMA,
        ],
    )
    out5 = sc(x4, tab_lin, pos)
    out = out5.transpose(0, 2, 4, 1, 3)
    return out.reshape(_B, _N, _D)


# speedup vs baseline: 3.9917x; 1.2311x over previous
import jax
import jax.numpy as jnp
from jax import lax
from jax.experimental import pallas as pl
from jax.experimental.pallas import tpu as pltpu
from jax.experimental.pallas import tpu_sc as plsc

_B, _N, _D = 1024, 1024, 32
_NC, _NS = 2, 16
_NW = _NC * _NS
_BLOCKS_PER_W = _B // _NW
_IDX_MINOR = 128
_HALF = _N // 2
_JP = _HALF // _IDX_MINOR
_NCHUNK = 2 * _BLOCKS_PER_W
_PITCH = 133


def _pos_body(g_ref, w_ref, b_ref, o_ref):
    o_ref[...] = (
        jnp.dot(g_ref[...], w_ref[...], preferred_element_type=jnp.float32)
        + b_ref[...]
    )


def _sc_body(x_hbm, tab_hbm, pos_hbm, out_hbm,
             idx0, idx1, rows0, rows1, tr0, tr1, pos_v,
             g0, g1, o0, o1, i0, i1):
    c = lax.axis_index("c")
    s = lax.axis_index("s")
    wid = s * _NC + c
    base = wid * _BLOCKS_PER_W
    idx = (idx0, idx1)
    rows = (rows0, rows1)
    tr = (tr0, tr1)
    gsem = (g0, g1)
    osem = (o0, o1)
    isem = (i0, i1)
    pltpu.sync_copy(pos_hbm, pos_v)
    lane = lax.iota(jnp.int32, 16)
    db_lo = lax.shift_right_logical(lane, 3)
    db_hi = db_lo + 2
    dr_c = lax.bitwise_and(lane, 7)

    def fetch_idx(t, buf):
        b = base + lax.shift_right_logical(t, 1)
        h = lax.bitwise_and(t, 1)
        bb = lax.shift_right_logical(b, 3)
        br = lax.bitwise_and(b, 7)
        pltpu.async_copy(
            x_hbm.at[bb, pl.ds(lax.shift_left(h, 2), _JP), br],
            idx[buf], isem[buf])

    def drain_idx(buf):
        pltpu.make_async_copy(
            x_hbm.at[0, pl.ds(0, _JP), 0], idx[buf], isem[buf]).wait()

    def fire(buf):
        for j in range(_JP):
            pltpu.async_copy(
                tab_hbm.at[idx[buf].at[j]],
                rows[buf].at[pl.ds(j * _IDX_MINOR, _IDX_MINOR)],
                gsem[buf],
            )

    out_dummy = out_hbm.at[0].at[:, pl.ds(0, 4)]
    fetch_idx(0, 0)
    drain_idx(0)
    fire(0)
    fetch_idx(1, 1)

    def pair(i, carry):
        for j in range(2):
            cur = j
            nxt = 1 - j
            t = i * 2 + j

            @pl.when(t + 1 < _NCHUNK)
            def _fire_next():
                drain_idx(nxt)
                fire(nxt)

            pltpu.make_async_copy(tab_hbm.at[pl.ds(0, _HALF)],
                                  rows[cur], gsem[cur]).wait()

            @pl.when(t + 2 < _NCHUNK)
            def _prefetch_idx():
                fetch_idx(t + 2, cur)

            @pl.when(t >= 2)
            def _wait_wb():
                pltpu.make_async_copy(
                    tr[cur].at[:, :, :, pl.ds(0, _IDX_MINOR)],
                    out_dummy, osem[cur]).wait()

            h = lax.bitwise_and(t, 1)
            noff = lax.shift_left(h, 9)

            @plsc.parallel_loop(0, _HALF, step=1, unroll=8)
            def transpose_add(k):
                nb = lax.shift_right_logical(k, 7)
                nc = lax.bitwise_and(k, 127)
                nbv = jnp.broadcast_to(nb, (16,))
                ncv = jnp.broadcast_to(nc, (16,))
                ng = noff + k
                v0 = rows[cur][k, pl.ds(0, 16)] + pos_v[ng, pl.ds(0, 16)]
                v1 = rows[cur][k, pl.ds(16, 16)] + pos_v[ng, pl.ds(16, 16)]
                plsc.store_scatter(tr[cur], [db_lo, nbv, dr_c, ncv], v0)
                plsc.store_scatter(tr[cur], [db_hi, nbv, dr_c, ncv], v1)

            b = base + lax.shift_right_logical(t, 1)
            pltpu.async_copy(
                tr[cur].at[:, :, :, pl.ds(0, _IDX_MINOR)],
                out_hbm.at[b].at[:, pl.ds(lax.shift_left(h, 2), 4)],
                osem[cur],
            )
        return carry

    lax.fori_loop(0, _NCHUNK // 2, pair, 0)
    pltpu.make_async_copy(tr[0].at[:, :, :, pl.ds(0, _IDX_MINOR)],
                          out_dummy, osem[0]).wait()
    pltpu.make_async_copy(tr[1].at[:, :, :, pl.ds(0, _IDX_MINOR)],
                          out_dummy, osem[1]).wait()


def kernel(x, tok_table, pos_W, pos_b, grid):
    g2 = grid.reshape(_N, 4)
    pos = pl.pallas_call(
        _pos_body,
        out_shape=jax.ShapeDtypeStruct((_N, _D), jnp.float32),
    )(g2, pos_W.T, pos_b.reshape(1, _D))

    x4 = x.reshape(_B // 8, 8, 8, _IDX_MINOR).transpose(0, 2, 1, 3)
    tab_lin = lax.optimization_barrier(tok_table.T).T
    sc = pl.kernel(
        _sc_body,
        out_type=jax.ShapeDtypeStruct((_B, 4, 8, 8, _IDX_MINOR), jnp.float32),
        mesh=plsc.VectorSubcoreMesh(core_axis_name="c", subcore_axis_name="s"),
        compiler_params=pltpu.CompilerParams(
            use_tc_tiling_on_sc=False, needs_layout_passes=False),
        scratch_types=[
            pltpu.VMEM((_JP, _IDX_MINOR), jnp.int32),
            pltpu.VMEM((_JP, _IDX_MINOR), jnp.int32),
            pltpu.VMEM((_HALF, _D), jnp.float32),
            pltpu.VMEM((_HALF, _D), jnp.float32),
            pltpu.VMEM((4, 4, 8, _PITCH), jnp.float32),
            pltpu.VMEM((4, 4, 8, _PITCH), jnp.float32),
            pltpu.VMEM((_N, _D), jnp.float32),
            pltpu.SemaphoreType.DMA,
            pltpu.SemaphoreType.DMA,
            pltpu.SemaphoreType.DMA,
            pltpu.SemaphoreType.DMA,
            pltpu.SemaphoreType.DMA,
            pltpu.SemaphoreType.DMA,
        ],
    )
    out5 = sc(x4, tab_lin, pos)
    out = out5.transpose(0, 2, 4, 1, 3)
    return out.reshape(_B, _N, _D)
